# baseline (speedup 1.0000x reference)
import functools

import jax
import jax.numpy as jnp
from jax import lax
from jax.experimental import pallas as pl
from jax.experimental.pallas import tpu as pltpu
from jax.experimental.pallas import tpu_sc as plsc

L = 16
NC, NS = 2, 16
NW = NC * NS
SC_CHUNK = 32
SB = 8


def _make_sc_kernel(B, S, N, D):
    S_W = S // NW

    def body(x_hbm, a_hbm, p_hbm, out_hbm, a_v, x_v, p_v, o_v):
        wid = lax.axis_index("s") * NC + lax.axis_index("c")
        base = wid * S_W
        pltpu.sync_copy(a_hbm, a_v)

        def chunk(ci, carry):
            s0 = base + ci * SC_CHUNK
            for bi in range(B):
                pltpu.sync_copy(
                    x_hbm.at[pl.ds((bi * S + s0) * N, SC_CHUNK * N)],
                    x_v.at[bi],
                )
            for sub in range(SC_CHUNK // SB):
                s1 = s0 + sub * SB
                pltpu.sync_copy(p_hbm.at[pl.ds(s1, SB)], p_v)
                for si in range(SB):
                    flat0 = (sub * SB + si) * N
                    w0 = (flat0 // L) * L
                    bvecs = []
                    for bi in range(B):
                        wnd = x_v[bi, pl.ds(w0, L)]
                        bvecs.append(
                            [jnp.full((L,), wnd[flat0 - w0 + ni]) for ni in range(N)]
                        )

                    def dblk(dci, c2):
                        for u in range(4):
                            off = (dci * 4 + u) * L
                            pvec = p_v[si, pl.ds(off, L)]
                            avecs = [a_v[ni, pl.ds(off, L)] for ni in range(N)]
                            for bi in range(B):
                                acc = pvec
                                for ni in range(N):
                                    acc = acc + bvecs[bi][ni] * avecs[ni]
                                o_v[bi, si, pl.ds(off, L)] = acc
                        return c2

                    lax.fori_loop(0, D // L // 4, dblk, 0)
                pltpu.sync_copy(o_v, out_hbm.at[:, pl.ds(s1, SB)])
            return carry

        lax.fori_loop(0, S_W // SC_CHUNK, chunk, 0)

    return pl.kernel(
        body,
        out_type=jax.ShapeDtypeStruct((B, S, D), jnp.float32),
        mesh=plsc.VectorSubcoreMesh(core_axis_name="c", subcore_axis_name="s"),
        scratch_types=[
            pltpu.VMEM((N, D), jnp.float32),
            pltpu.VMEM((B, SC_CHUNK * N), jnp.float32),
            pltpu.VMEM((SB, D), jnp.float32),
            pltpu.VMEM((B, SB, D), jnp.float32),
        ],
    )


@jax.jit
def kernel(x, allele_embedding, position_embedding):
    B, S, N = x.shape
    D = allele_embedding.shape[1]
    sc = _make_sc_kernel(B, S, N, D)
    return sc(x.reshape(B * S * N), allele_embedding, position_embedding)


# speedup vs baseline: 1.3784x; 1.3784x over previous
import functools

import jax
import jax.numpy as jnp
from jax import lax
from jax.experimental import pallas as pl
from jax.experimental.pallas import tpu as pltpu
from jax.experimental.pallas import tpu_sc as plsc

L = 16
NC, NS = 2, 16
NW = NC * NS
SB = 8


def _make_sc_kernel(B, S, N, D):
    S_W = S // NW
    T = S_W // SB

    def body(x_hbm, a_hbm, p_hbm, out_hbm,
             a_v, x_v, p_v0, p_v1, o_v0, o_v1,
             sx, sp0, sp1, so0, so1):
        wid = lax.axis_index("s") * NC + lax.axis_index("c")
        base = wid * S_W

        p_bufs = (p_v0, p_v1)
        p_sems = (sp0, sp1)
        o_bufs = (o_v0, o_v1)
        o_sems = (so0, so1)

        def start_p(t, slot):
            pltpu.async_copy(p_hbm.at[pl.ds(base + t * SB, SB)],
                             p_bufs[slot], p_sems[slot])

        def wait_p(slot):
            pltpu.make_async_copy(p_hbm.at[pl.ds(0, SB)],
                                  p_bufs[slot], p_sems[slot]).wait()

        def start_o(t, slot):
            pltpu.async_copy(o_bufs[slot],
                             out_hbm.at[:, pl.ds(base + t * SB, SB)],
                             o_sems[slot])

        def wait_o(slot):
            pltpu.make_async_copy(o_bufs[slot],
                                  out_hbm.at[:, pl.ds(0, SB)],
                                  o_sems[slot]).wait()

        for bi in range(B):
            pltpu.async_copy(
                x_hbm.at[pl.ds((bi * S + base) * N, S_W * N)],
                x_v.at[bi], sx)
        start_p(0, 0)
        start_p(1, 1)
        pltpu.sync_copy(a_hbm, a_v)
        for bi in range(B):
            pltpu.make_async_copy(x_hbm.at[pl.ds(0, S_W * N)],
                                  x_v.at[bi], sx).wait()

        def compute(t_static_base, t, slot):
            p_v = p_bufs[slot]
            o_v = o_bufs[slot]
            for si in range(SB):
                flat0 = (t_static_base * SB + si) * N
                w0 = (flat0 // L) * L
                lane0 = flat0 - w0
                woff = (t - t_static_base) * SB * N
                bvecs = []
                for bi in range(B):
                    wnd = x_v[bi, pl.ds(woff + w0, L)]
                    bvecs.append(
                        [jnp.full((L,), wnd[lane0 + ni]) for ni in range(N)])

                def dblk(dci, c2):
                    for u in range(2):
                        off = (dci * 2 + u) * L
                        pvec = p_v[si, pl.ds(off, L)]
                        avecs = [a_v[ni, pl.ds(off, L)] for ni in range(N)]
                        for bi in range(B):
                            acc = pvec + bvecs[bi][0] * avecs[0]
                            for ni in range(1, N):
                                acc = acc + bvecs[bi][ni] * avecs[ni]
                            o_v[bi, si, pl.ds(off, L)] = acc
                    return c2

                lax.fori_loop(0, D // L // 2, dblk, 0)

        wait_p(0)
        compute(0, 0, 0)
        start_o(0, 0)
        start_p(2, 0)
        wait_p(1)
        compute(1, 1, 1)
        start_o(1, 1)
        start_p(3, 1)

        def pair(t2, carry):
            t = t2 * 2
            wait_p(0)
            wait_o(0)
            compute(0, t, 0)
            start_o(t, 0)
            start_p(t + 2, 0)
            wait_p(1)
            wait_o(1)
            compute(1, t + 1, 1)
            start_o(t + 1, 1)
            start_p(t + 3, 1)
            return carry

        lax.fori_loop(1, T // 2 - 1, pair, 0)

        wait_p(0)
        wait_o(0)
        compute(0, T - 2, 0)
        start_o(T - 2, 0)
        wait_p(1)
        wait_o(1)
        compute(1, T - 1, 1)
        start_o(T - 1, 1)
        wait_o(0)
        wait_o(1)

    return pl.kernel(
        body,
        out_type=jax.ShapeDtypeStruct((B, S, D), jnp.float32),
        mesh=plsc.VectorSubcoreMesh(core_axis_name="c", subcore_axis_name="s"),
        scratch_types=[
            pltpu.VMEM((N, D), jnp.float32),
            pltpu.VMEM((B, S_W * N), jnp.float32),
            pltpu.VMEM((SB, D), jnp.float32),
            pltpu.VMEM((SB, D), jnp.float32),
            pltpu.VMEM((B, SB, D), jnp.float32),
            pltpu.VMEM((B, SB, D), jnp.float32),
            pltpu.SemaphoreType.DMA,
            pltpu.SemaphoreType.DMA,
            pltpu.SemaphoreType.DMA,
            pltpu.SemaphoreType.DMA,
            pltpu.SemaphoreType.DMA,
        ],
    )


@jax.jit
def kernel(x, allele_embedding, position_embedding):
    B, S, N = x.shape
    D = allele_embedding.shape[1]
    sc = _make_sc_kernel(B, S, N, D)
    return sc(x.reshape(B * S * N), allele_embedding, position_embedding)


# speedup vs baseline: 5.1996x; 3.7723x over previous
import jax
import jax.numpy as jnp
from jax.experimental import pallas as pl

S_BLK = 1024


def _geno_block(x_ref, a_ref, p_ref, o_ref):
    p = p_ref[...]
    a = a_ref[...]
    x = x_ref[...]
    for bi in range(x.shape[0]):
        y = jnp.dot(x[bi], a, preferred_element_type=jnp.float32)
        o_ref[bi] = y + p


@jax.jit
def kernel(x, allele_embedding, position_embedding):
    B, S, N = x.shape
    D = allele_embedding.shape[1]
    grid = (S // S_BLK,)
    out = pl.pallas_call(
        _geno_block,
        grid=grid,
        in_specs=[
            pl.BlockSpec((B, S_BLK, N), lambda i: (0, i, 0)),
            pl.BlockSpec((N, D), lambda i: (0, 0)),
            pl.BlockSpec((S_BLK, D), lambda i: (i, 0)),
        ],
        out_specs=pl.BlockSpec((B, S_BLK, D), lambda i: (0, i, 0)),
        out_shape=jax.ShapeDtypeStruct((B, S, D), jnp.float32),
    )(x, allele_embedding, position_embedding)
    return out


# speedup vs baseline: 5.2167x; 1.0033x over previous
import jax
import jax.numpy as jnp
from jax.experimental import pallas as pl

S_BLK = 1280


def _geno_block(x_ref, a_ref, p_ref, o_ref):
    p = p_ref[...]
    a = a_ref[...]
    x = x_ref[...]
    for bi in range(x.shape[0]):
        y = jnp.dot(x[bi], a, preferred_element_type=jnp.float32)
        o_ref[bi] = y + p


@jax.jit
def kernel(x, allele_embedding, position_embedding):
    B, S, N = x.shape
    D = allele_embedding.shape[1]
    grid = ((S + S_BLK - 1) // S_BLK,)
    out = pl.pallas_call(
        _geno_block,
        grid=grid,
        in_specs=[
            pl.BlockSpec((B, S_BLK, N), lambda i: (0, i, 0)),
            pl.BlockSpec((N, D), lambda i: (0, 0)),
            pl.BlockSpec((S_BLK, D), lambda i: (i, 0)),
        ],
        out_specs=pl.BlockSpec((B, S_BLK, D), lambda i: (0, i, 0)),
        out_shape=jax.ShapeDtypeStruct((B, S, D), jnp.float32),
    )(x, allele_embedding, position_embedding)
    return out


# speedup vs baseline: 6.5894x; 1.2631x over previous
import jax
import jax.numpy as jnp
from jax.experimental import pallas as pl

S_BLK = 1408


def _geno_block(xt_ref, a_ref, p_ref, o_ref):
    p = p_ref[...]
    a = a_ref[...]
    xt = xt_ref[...]
    for bi in range(xt.shape[0]):
        y = jax.lax.dot_general(
            xt[bi], a,
            dimension_numbers=(((0,), (0,)), ((), ())),
            preferred_element_type=jnp.float32,
        )
        o_ref[bi] = y + p


@jax.jit
def kernel(x, allele_embedding, position_embedding):
    B, S, N = x.shape
    D = allele_embedding.shape[1]
    xt = x.transpose(0, 2, 1)
    grid = ((S + S_BLK - 1) // S_BLK,)
    out = pl.pallas_call(
        _geno_block,
        grid=grid,
        in_specs=[
            pl.BlockSpec((B, N, S_BLK), lambda i: (0, 0, i)),
            pl.BlockSpec((N, D), lambda i: (0, 0)),
            pl.BlockSpec((S_BLK, D), lambda i: (i, 0)),
        ],
        out_specs=pl.BlockSpec((B, S_BLK, D), lambda i: (0, i, 0)),
        out_shape=jax.ShapeDtypeStruct((B, S, D), jnp.float32),
    )(xt, allele_embedding, position_embedding)
    return out
